# Initial kernel scaffold; baseline (speedup 1.0000x reference)
import jax
import jax.numpy as jnp
from jax.experimental import pallas as pl


def kernel(x, w1, b1, w2, b2, we0, we1, we2):
    raise NotImplementedError("write your pallas kernel here")


# speedup vs baseline: 3.0174x; 3.0174x over previous
import math

import jax
import jax.numpy as jnp
import numpy as np
from jax.experimental import pallas as pl
from jax.experimental.pallas import tpu as pltpu

_PATCH_LENS = (8, 16, 32)
_D_MODEL = 1024
_REGION = 32
_TPN = 4
_TM = 128


def _make_pe_np(d_model: int, length: int) -> np.ndarray:
    position = np.arange(length, dtype=np.float32)[:, None]
    div_term = np.exp(
        np.arange(0, d_model, 2, dtype=np.float32) * -(math.log(10000.0) / d_model)
    )
    pe = np.zeros((length, d_model), dtype=np.float32)
    pe[:, 0::2] = np.sin(position * div_term)
    pe[:, 1::2] = np.cos(position * div_term)
    return pe


def _body(x_ref, w1_ref, b1_ref, w2_ref, b2_ref, wbig_ref, pe_ref, o_ref):
    xb = x_ref[...]
    h = jnp.dot(xb, w1_ref[...], preferred_element_type=jnp.float32)
    h = jnp.maximum(h + b1_ref[...], 0.0)
    logits = jnp.dot(h, w2_ref[...], preferred_element_type=jnp.float32)
    logits = logits + b2_ref[...]
    l0 = logits[:, 0:1]
    l1 = logits[:, 1:2]
    l2 = logits[:, 2:3]
    m0 = jnp.logical_and(l0 >= l1, l0 >= l2)
    m1 = jnp.logical_and(jnp.logical_not(m0), l1 >= l2)
    m2 = jnp.logical_not(jnp.logical_or(m0, m1))
    zero = jnp.zeros_like(xb)
    xe = jnp.concatenate(
        [
            jnp.where(m0, xb, zero),
            jnp.where(m1, xb, zero),
            jnp.where(m2, xb, zero),
        ],
        axis=1,
    )
    acc = jnp.dot(xe, wbig_ref[...], preferred_element_type=jnp.float32)
    o_ref[...] = acc + pe_ref[...]


def kernel(x, w1, b1, w2, b2, we0, we1, we2):
    n_batch, n_ch, seq_len = x.shape
    n_rows = n_batch * n_ch
    n_regions = seq_len // _REGION
    n_tokens = n_rows * n_regions
    d = we0.shape[1]
    cols = _TPN * d

    xt = x.reshape(n_tokens, _REGION)

    zc = jnp.zeros((we1.shape[0], d), dtype=x.dtype)
    w0big = jax.scipy.linalg.block_diag(we0, we0, we0, we0)
    w1big = jnp.concatenate(
        [
            jnp.concatenate([we1, we1, we1, zc], axis=1),
            jnp.concatenate([zc, zc, zc, we1], axis=1),
        ],
        axis=0,
    )
    w2big = jnp.tile(we2, (1, _TPN))
    wbig = jnp.concatenate([w0big, w1big, w2big], axis=0)

    w2p = jnp.zeros((w2.shape[0], 128), dtype=w2.dtype).at[:, :3].set(w2)
    b2p = jnp.zeros((1, 128), dtype=b2.dtype).at[:, :3].set(b2)
    b1r = b1.reshape(1, -1)

    pe = _make_pe_np(d, n_regions * _TPN).reshape(n_regions, cols)
    reps = _TM // (_TM if n_regions >= _TM else n_regions)
    pe_blk = jnp.asarray(np.tile(pe, (max(reps, 1), 1))[:_TM])

    grid = (n_tokens // _TM,)
    out = pl.pallas_call(
        _body,
        grid=grid,
        in_specs=[
            pl.BlockSpec((_TM, _REGION), lambda i: (i, 0)),
            pl.BlockSpec(w1.shape, lambda i: (0, 0)),
            pl.BlockSpec((1, b1.shape[0]), lambda i: (0, 0)),
            pl.BlockSpec((w2.shape[0], 128), lambda i: (0, 0)),
            pl.BlockSpec((1, 128), lambda i: (0, 0)),
            pl.BlockSpec((3 * _REGION, cols), lambda i: (0, 0)),
            pl.BlockSpec((_TM, cols), lambda i: (0, 0)),
        ],
        out_specs=pl.BlockSpec((_TM, cols), lambda i: (i, 0)),
        out_shape=jax.ShapeDtypeStruct((n_tokens, cols), x.dtype),
        compiler_params=pltpu.CompilerParams(
            dimension_semantics=("parallel",),
        ),
    )(xt, w1, b1r, w2p, b2p, wbig, pe_blk)

    return out.reshape(n_rows, n_regions * _TPN, d)


# speedup vs baseline: 3.0236x; 1.0021x over previous
import math

import jax
import jax.numpy as jnp
import numpy as np
from jax.experimental import pallas as pl
from jax.experimental.pallas import tpu as pltpu

_PATCH_LENS = (8, 16, 32)
_D_MODEL = 1024
_REGION = 32
_TPN = 4
_TM = 128


def _make_pe_np(d_model: int, length: int) -> np.ndarray:
    position = np.arange(length, dtype=np.float32)[:, None]
    div_term = np.exp(
        np.arange(0, d_model, 2, dtype=np.float32) * -(math.log(10000.0) / d_model)
    )
    pe = np.zeros((length, d_model), dtype=np.float32)
    pe[:, 0::2] = np.sin(position * div_term)
    pe[:, 1::2] = np.cos(position * div_term)
    return pe


def _body(x_ref, w1_ref, b1_ref, w2_ref, b2_ref, wbig_ref, pe_ref, o_ref):
    xb = x_ref[...]
    h = jnp.dot(xb, w1_ref[...], preferred_element_type=jnp.float32)
    h = jnp.maximum(h + b1_ref[...], 0.0)
    logits = jnp.dot(h, w2_ref[...], preferred_element_type=jnp.float32)
    logits = logits + b2_ref[...]
    l0 = logits[:, 0:1]
    l1 = logits[:, 1:2]
    l2 = logits[:, 2:3]
    m0 = jnp.logical_and(l0 >= l1, l0 >= l2)
    m1 = jnp.logical_and(jnp.logical_not(m0), l1 >= l2)
    m2 = jnp.logical_not(jnp.logical_or(m0, m1))
    zero = jnp.zeros_like(xb)
    xe = jnp.concatenate(
        [
            jnp.where(m0, xb, zero),
            jnp.where(m1, xb, zero),
            jnp.where(m2, xb, zero),
        ],
        axis=1,
    )
    acc = jnp.dot(
        xe.astype(jnp.bfloat16), wbig_ref[...],
        preferred_element_type=jnp.float32,
    )
    o_ref[...] = acc + pe_ref[...]


def kernel(x, w1, b1, w2, b2, we0, we1, we2):
    n_batch, n_ch, seq_len = x.shape
    n_rows = n_batch * n_ch
    n_regions = seq_len // _REGION
    n_tokens = n_rows * n_regions
    d = we0.shape[1]
    cols = _TPN * d

    xt = x.reshape(n_tokens, _REGION)

    zc = jnp.zeros((we1.shape[0], d), dtype=x.dtype)
    w0big = jax.scipy.linalg.block_diag(we0, we0, we0, we0)
    w1big = jnp.concatenate(
        [
            jnp.concatenate([we1, we1, we1, zc], axis=1),
            jnp.concatenate([zc, zc, zc, we1], axis=1),
        ],
        axis=0,
    )
    w2big = jnp.tile(we2, (1, _TPN))
    wbig = jnp.concatenate([w0big, w1big, w2big], axis=0).astype(jnp.bfloat16)

    w2p = jnp.zeros((w2.shape[0], 128), dtype=w2.dtype).at[:, :3].set(w2)
    b2p = jnp.zeros((1, 128), dtype=b2.dtype).at[:, :3].set(b2)
    b1r = b1.reshape(1, -1)

    pe = _make_pe_np(d, n_regions * _TPN).reshape(n_regions, cols)
    reps = _TM // (_TM if n_regions >= _TM else n_regions)
    pe_blk = jnp.asarray(np.tile(pe, (max(reps, 1), 1))[:_TM])

    grid = (n_tokens // _TM,)
    out = pl.pallas_call(
        _body,
        grid=grid,
        in_specs=[
            pl.BlockSpec((_TM, _REGION), lambda i: (i, 0)),
            pl.BlockSpec(w1.shape, lambda i: (0, 0)),
            pl.BlockSpec((1, b1.shape[0]), lambda i: (0, 0)),
            pl.BlockSpec((w2.shape[0], 128), lambda i: (0, 0)),
            pl.BlockSpec((1, 128), lambda i: (0, 0)),
            pl.BlockSpec((3 * _REGION, cols), lambda i: (0, 0)),
            pl.BlockSpec((_TM, cols), lambda i: (0, 0)),
        ],
        out_specs=pl.BlockSpec((_TM, cols), lambda i: (i, 0)),
        out_shape=jax.ShapeDtypeStruct((n_tokens, cols), x.dtype),
        compiler_params=pltpu.CompilerParams(
            dimension_semantics=("parallel",),
        ),
    )(xt, w1, b1r, w2p, b2p, wbig, pe_blk)

    return out.reshape(n_rows, n_regions * _TPN, d)


# speedup vs baseline: 3.3652x; 1.1130x over previous
import math

import jax
import jax.numpy as jnp
import numpy as np
from jax.experimental import pallas as pl
from jax.experimental.pallas import tpu as pltpu

_PATCH_LENS = (8, 16, 32)
_D_MODEL = 1024
_REGION = 32
_TPN = 4
_TM = 256


def _make_pe_np(d_model: int, length: int) -> np.ndarray:
    position = np.arange(length, dtype=np.float32)[:, None]
    div_term = np.exp(
        np.arange(0, d_model, 2, dtype=np.float32) * -(math.log(10000.0) / d_model)
    )
    pe = np.zeros((length, d_model), dtype=np.float32)
    pe[:, 0::2] = np.sin(position * div_term)
    pe[:, 1::2] = np.cos(position * div_term)
    return pe


def _body(x_ref, w1_ref, b1_ref, w2_ref, b2_ref, wbig_ref, pe_ref, o_ref):
    xb = x_ref[...]
    h = jnp.dot(xb, w1_ref[...], preferred_element_type=jnp.float32)
    h = jnp.maximum(h + b1_ref[...], 0.0)
    logits = jnp.dot(h, w2_ref[...], preferred_element_type=jnp.float32)
    logits = logits + b2_ref[...]
    l0 = logits[:, 0:1]
    l1 = logits[:, 1:2]
    l2 = logits[:, 2:3]
    m0 = jnp.logical_and(l0 >= l1, l0 >= l2)
    m1 = jnp.logical_and(jnp.logical_not(m0), l1 >= l2)
    m2 = jnp.logical_not(jnp.logical_or(m0, m1))
    zero = jnp.zeros_like(xb)
    xe = jnp.concatenate(
        [
            jnp.where(m0, xb, zero),
            jnp.where(m1, xb, zero),
            jnp.where(m2, xb, zero),
        ],
        axis=1,
    )
    acc = jnp.dot(
        xe.astype(jnp.bfloat16), wbig_ref[...],
        preferred_element_type=jnp.float32,
    )
    o_ref[...] = acc + pe_ref[...]


def kernel(x, w1, b1, w2, b2, we0, we1, we2):
    n_batch, n_ch, seq_len = x.shape
    n_rows = n_batch * n_ch
    n_regions = seq_len // _REGION
    n_tokens = n_rows * n_regions
    d = we0.shape[1]
    cols = _TPN * d

    xt = x.reshape(n_tokens, _REGION)

    zc = jnp.zeros((we1.shape[0], d), dtype=x.dtype)
    w0big = jax.scipy.linalg.block_diag(we0, we0, we0, we0)
    w1big = jnp.concatenate(
        [
            jnp.concatenate([we1, we1, we1, zc], axis=1),
            jnp.concatenate([zc, zc, zc, we1], axis=1),
        ],
        axis=0,
    )
    w2big = jnp.tile(we2, (1, _TPN))
    wbig = jnp.concatenate([w0big, w1big, w2big], axis=0).astype(jnp.bfloat16)

    w2p = jnp.zeros((w2.shape[0], 128), dtype=w2.dtype).at[:, :3].set(w2)
    b2p = jnp.zeros((1, 128), dtype=b2.dtype).at[:, :3].set(b2)
    b1r = b1.reshape(1, -1)

    pe = _make_pe_np(d, n_regions * _TPN).reshape(n_regions, cols)
    reps = _TM // (_TM if n_regions >= _TM else n_regions)
    pe_blk = jnp.asarray(np.tile(pe, (max(reps, 1), 1))[:_TM])

    grid = (n_tokens // _TM,)
    out = pl.pallas_call(
        _body,
        grid=grid,
        in_specs=[
            pl.BlockSpec((_TM, _REGION), lambda i: (i, 0)),
            pl.BlockSpec(w1.shape, lambda i: (0, 0)),
            pl.BlockSpec((1, b1.shape[0]), lambda i: (0, 0)),
            pl.BlockSpec((w2.shape[0], 128), lambda i: (0, 0)),
            pl.BlockSpec((1, 128), lambda i: (0, 0)),
            pl.BlockSpec((3 * _REGION, cols), lambda i: (0, 0)),
            pl.BlockSpec((_TM, cols), lambda i: (0, 0)),
        ],
        out_specs=pl.BlockSpec((_TM, cols), lambda i: (i, 0)),
        out_shape=jax.ShapeDtypeStruct((n_tokens, cols), x.dtype),
        compiler_params=pltpu.CompilerParams(
            dimension_semantics=("parallel",),
        ),
    )(xt, w1, b1r, w2p, b2p, wbig, pe_blk)

    return out.reshape(n_rows, n_regions * _TPN, d)
